# baseline (speedup 1.0000x reference)
import functools
import math

import jax
import jax.numpy as jnp
from jax import lax
from jax.experimental import pallas as pl
from jax.experimental.pallas import tpu as pltpu
from jax.experimental.pallas import tpu_sc as plsc

_V = 8192
_N = 8192
_SMOOTH = 0.1
_CONF = 1.0 - _SMOOTH
_PAD = 1
_EPS = _SMOOTH / (_V - 1)
_C = _CONF * math.log(_CONF) + _SMOOTH * math.log(_EPS)
_BLK = 256

_NW = 32
_CHUNK = _N // _NW
_L = 16


def _tc_kernel(t_ref, x_ref, sum_ref):
    i = pl.program_id(0)

    @pl.when(i == 0)
    def _():
        sum_ref[0, 0] = 0.0

    xb = x_ref[...]
    t = t_ref[0, 0, :]
    m = jnp.max(xb, axis=1, keepdims=True)
    s = jnp.sum(jnp.exp(xb - m), axis=1)
    lse = m[:, 0] + jnp.log(s)
    sum_x = jnp.sum(xb, axis=1)
    idx = jax.lax.broadcasted_iota(jnp.int32, xb.shape, 1)
    x_t = jnp.sum(jnp.where(idx == t[:, None], xb, 0.0), axis=1)
    keep = (t != _PAD).astype(jnp.float32)
    per = _C + lse - _EPS * sum_x - (_CONF - _EPS) * x_t
    sum_ref[0, 0] += jnp.sum(per * keep)


def _sc_kernel(t_hbm, cnt_out, t_v, sc_v, sem):
    wid = lax.axis_index("s") * 2 + lax.axis_index("c")
    base = wid * _CHUNK
    pltpu.sync_copy(t_hbm.at[pl.ds(base, _CHUNK)], t_v)
    cnt = jnp.zeros((_L,), jnp.float32)
    for j in range(_CHUNK // _L):
        keep = t_v[pl.ds(j * _L, _L)] != _PAD
        cnt = cnt + jnp.where(keep, 1.0, 0.0)
    sc_v[...] = cnt
    pltpu.sync_copy(sc_v, cnt_out.at[pl.ds(wid * _L, _L)])


@jax.jit
def kernel(x, target):
    xf = x.reshape(-1, _V)
    n = xf.shape[0]
    nblk = n // _BLK
    t32 = target.reshape(-1).astype(jnp.int32)
    t3d = t32.reshape(nblk, 1, _BLK)

    mesh = plsc.VectorSubcoreMesh(core_axis_name="c", subcore_axis_name="s")
    sc_call = functools.partial(
        pl.kernel, mesh=mesh,
        out_type=jax.ShapeDtypeStruct((_NW * _L,), jnp.float32),
        scratch_types=[
            pltpu.VMEM((_CHUNK,), jnp.int32),
            pltpu.VMEM((_L,), jnp.float32),
            pltpu.SemaphoreType.DMA,
        ],
    )(_sc_kernel)
    cnt_parts = sc_call(t32)

    tc_sum = pl.pallas_call(
        _tc_kernel,
        grid=(nblk,),
        in_specs=[
            pl.BlockSpec((1, 1, _BLK), lambda i: (i, 0, 0)),
            pl.BlockSpec((_BLK, _V), lambda i: (i, 0)),
        ],
        out_specs=pl.BlockSpec(memory_space=pltpu.SMEM),
        out_shape=jax.ShapeDtypeStruct((1, 1), jnp.float32),
    )(t3d, xf)

    return tc_sum[0, 0] / jnp.sum(cnt_parts)


# speedup vs baseline: 1.2593x; 1.2593x over previous
import math

import jax
import jax.numpy as jnp
from jax.experimental import pallas as pl
from jax.experimental.pallas import tpu as pltpu

_V = 8192
_SMOOTH = 0.1
_CONF = 1.0 - _SMOOTH
_PAD = 1
_EPS = _SMOOTH / (_V - 1)
_C = _CONF * math.log(_CONF) + _SMOOTH * math.log(_EPS)
_BLK = 512


def _loss_kernel(t_ref, x_ref, sum_ref, cnt_ref):
    i = pl.program_id(0)

    @pl.when(i == 0)
    def _():
        sum_ref[0, 0] = 0.0
        cnt_ref[0, 0] = 0.0

    xb = x_ref[...]
    t = t_ref[0, 0, :]
    m = jnp.max(xb, axis=1, keepdims=True)
    s = jnp.sum(jnp.exp(xb - m), axis=1)
    lse = m[:, 0] + jnp.log(s)
    sum_x = jnp.sum(xb, axis=1)
    idx = jax.lax.broadcasted_iota(jnp.int32, xb.shape, 1)
    x_t = jnp.sum(jnp.where(idx == t[:, None], xb, 0.0), axis=1)
    keep = (t != _PAD).astype(jnp.float32)
    per = _C + lse - _EPS * sum_x - (_CONF - _EPS) * x_t
    sum_ref[0, 0] += jnp.sum(per * keep)
    cnt_ref[0, 0] += jnp.sum(keep)


@jax.jit
def kernel(x, target):
    xf = x.reshape(-1, _V)
    n = xf.shape[0]
    nblk = n // _BLK
    t = target.reshape(-1).astype(jnp.int32).reshape(nblk, 1, _BLK)
    loss_sum, cnt = pl.pallas_call(
        _loss_kernel,
        grid=(nblk,),
        in_specs=[
            pl.BlockSpec((1, 1, _BLK), lambda i: (i, 0, 0)),
            pl.BlockSpec((_BLK, _V), lambda i: (i, 0)),
        ],
        out_specs=[
            pl.BlockSpec(memory_space=pltpu.SMEM),
            pl.BlockSpec(memory_space=pltpu.SMEM),
        ],
        out_shape=[
            jax.ShapeDtypeStruct((1, 1), jnp.float32),
            jax.ShapeDtypeStruct((1, 1), jnp.float32),
        ],
    )(t, xf)
    return loss_sum[0, 0] / cnt[0, 0]


# speedup vs baseline: 1.2875x; 1.0224x over previous
import math

import jax
import jax.numpy as jnp
from jax.experimental import pallas as pl
from jax.experimental.pallas import tpu as pltpu

_V = 8192
_SMOOTH = 0.1
_CONF = 1.0 - _SMOOTH
_PAD = 1
_EPS = _SMOOTH / (_V - 1)
_C = _CONF * math.log(_CONF) + _SMOOTH * math.log(_EPS)
_BLK = 512


def _loss_kernel(t_ref, x_ref, sum_ref, cnt_ref):
    i = pl.program_id(0)

    @pl.when(i == 0)
    def _():
        sum_ref[0, 0] = 0.0
        cnt_ref[0, 0] = 0.0

    xb = x_ref[...]
    t = t_ref[0, 0, :]
    m = jnp.max(xb, axis=1, keepdims=True)
    y = xb - m
    s = jnp.sum(jnp.exp(y), axis=1)
    idx = jax.lax.broadcasted_iota(jnp.int32, xb.shape, 1)
    w = jnp.where(idx == t[:, None], _CONF, _EPS)
    wsum = jnp.sum(y * w, axis=1)
    keep = (t != _PAD).astype(jnp.float32)
    per = _C + jnp.log(s) - wsum
    sum_ref[0, 0] += jnp.sum(per * keep)
    cnt_ref[0, 0] += jnp.sum(keep)


@jax.jit
def kernel(x, target):
    xf = x.reshape(-1, _V)
    n = xf.shape[0]
    nblk = n // _BLK
    t = target.reshape(-1).astype(jnp.int32).reshape(nblk, 1, _BLK)
    loss_sum, cnt = pl.pallas_call(
        _loss_kernel,
        grid=(nblk,),
        in_specs=[
            pl.BlockSpec((1, 1, _BLK), lambda i: (i, 0, 0)),
            pl.BlockSpec((_BLK, _V), lambda i: (i, 0)),
        ],
        out_specs=[
            pl.BlockSpec(memory_space=pltpu.SMEM),
            pl.BlockSpec(memory_space=pltpu.SMEM),
        ],
        out_shape=[
            jax.ShapeDtypeStruct((1, 1), jnp.float32),
            jax.ShapeDtypeStruct((1, 1), jnp.float32),
        ],
    )(t, xf)
    return loss_sum[0, 0] / cnt[0, 0]
